# Initial kernel scaffold; baseline (speedup 1.0000x reference)
import jax
import jax.numpy as jnp
from jax.experimental import pallas as pl


def kernel(x, edge_index, batch, c1_W1, c1_b1, c1_W2, c1_b2, c2_W1, c2_b1, c2_W2, c2_b2, c3_W1, c3_b1, c3_W2, c3_b2, c4_W1, c4_b1, c4_W2, c4_b2, c5_W1, c5_b1, c5_W2, c5_b2, l1, l2, l3, l4, l5):
    raise NotImplementedError("write your pallas kernel here")


# speedup vs baseline: 10.0267x; 10.0267x over previous
import functools

import jax
import jax.numpy as jnp
from jax import lax
from jax.experimental import pallas as pl
from jax.experimental.pallas import tpu as pltpu
from jax.experimental.pallas import tpu_sc as plsc

N = 10000
E = 320000
F_IN = 128
DIM = 64
OUT = 32
G = 128

NC = 2
NS = 16
NW = NC * NS
CH = 128
NCHUNK = 80
E_PAD = NW * NCHUNK * CH
Z_ROWS = 10016
ROW_BLK = 2504
ZSL = 632
ZSL_LAST = N - (NS - 1) * ZSL


def _sc_agg_body(z_hbm, srcw_hbm, dstw_hbm, zeros_hbm, out_hbm,
                 src_v, dst_v, row_a, row_b, acc, sem_a, sem_b):
    c = lax.axis_index("c")
    s = lax.axis_index("s")
    w = c * NS + s

    pltpu.sync_copy(srcw_hbm.at[pl.ds(w * NCHUNK, NCHUNK)], src_v)
    pltpu.sync_copy(dstw_hbm.at[pl.ds(w * NCHUNK, NCHUNK)], dst_v)

    @pl.when(s < NS - 1)
    def _():
        pltpu.sync_copy(zeros_hbm, acc.at[pl.ds(s * ZSL, ZSL)])

    @pl.when(s == NS - 1)
    def _():
        pltpu.sync_copy(zeros_hbm.at[pl.ds(0, ZSL_LAST)],
                        acc.at[pl.ds((NS - 1) * ZSL, ZSL_LAST)])

    plsc.subcore_barrier()

    def body(i, carry):
        j = 2 * i
        cp_a = pltpu.async_copy(z_hbm.at[src_v.at[j]], row_a, sem_a)
        cp_b = pltpu.async_copy(z_hbm.at[src_v.at[j + 1]], row_b, sem_b)
        cp_a.wait()
        pltpu.sync_copy(row_a, acc.at[dst_v.at[j]], add=True)
        cp_b.wait()
        pltpu.sync_copy(row_b, acc.at[dst_v.at[j + 1]], add=True)
        return carry

    lax.fori_loop(0, NCHUNK // 2, body, 0)
    plsc.subcore_barrier()

    @pl.when(s < NS - 1)
    def _():
        pltpu.sync_copy(acc.at[pl.ds(s * ZSL, ZSL)],
                        out_hbm.at[c, pl.ds(s * ZSL, ZSL)])

    @pl.when(s == NS - 1)
    def _():
        pltpu.sync_copy(acc.at[pl.ds((NS - 1) * ZSL, ZSL_LAST)],
                        out_hbm.at[c, pl.ds((NS - 1) * ZSL, ZSL_LAST)])


@functools.cache
def _sc_agg_kernel():
    return pl.kernel(
        _sc_agg_body,
        out_type=jax.ShapeDtypeStruct((NC, Z_ROWS, DIM), jnp.float32),
        mesh=plsc.VectorSubcoreMesh(core_axis_name="c", subcore_axis_name="s",
                                    num_cores=NC, num_subcores=NS),
        scratch_types=[
            pltpu.VMEM((NCHUNK, CH), jnp.int32),
            pltpu.VMEM((NCHUNK, CH), jnp.int32),
            pltpu.VMEM((CH, DIM), jnp.float32),
            pltpu.VMEM((CH, DIM), jnp.float32),
            pltpu.VMEM_SHARED((N, DIM), jnp.float32),
            pltpu.SemaphoreType.DMA,
            pltpu.SemaphoreType.DMA,
        ],
        compiler_params=pltpu.CompilerParams(use_tc_tiling_on_sc=False),
    )


def _sc_agg(z, srcw, dstw, zeros):
    return _sc_agg_kernel()(z, srcw, dstw, zeros)


def _pre_body(x_ref, w_ref, o_ref):
    o_ref[...] = jnp.dot(x_ref[...], w_ref[...],
                         preferred_element_type=jnp.float32)


def _tc_pre(x, w1):
    return pl.pallas_call(
        _pre_body,
        grid=(Z_ROWS // ROW_BLK,),
        in_specs=[
            pl.BlockSpec((ROW_BLK, F_IN), lambda i: (i, 0)),
            pl.BlockSpec((F_IN, DIM), lambda i: (0, 0)),
        ],
        out_specs=pl.BlockSpec((ROW_BLK, DIM), lambda i: (i, 0)),
        out_shape=jax.ShapeDtypeStruct((Z_ROWS, DIM), jnp.float32),
    )(x, w1)


def _tc_layer(z, agg, q, b1, w2, b2, li, w1n):
    has_next = w1n is not None
    if not has_next:
        w1n = w2
    out_shape = [jax.ShapeDtypeStruct((Z_ROWS, DIM), jnp.float32),
                 jax.ShapeDtypeStruct((Z_ROWS, OUT), jnp.float32)]
    out_specs = [pl.BlockSpec((ROW_BLK, DIM), lambda i: (i, 0)),
                 pl.BlockSpec((ROW_BLK, OUT), lambda i: (i, 0))]
    if not has_next:
        out_shape = out_shape[1:]
        out_specs = out_specs[1:]

    def body(z_ref, agg_ref, q_ref, b1_ref, w2_ref, b2_ref, li_ref, w1n_ref,
             *o_refs):
        if has_next:
            zn_ref, qn_ref = o_refs
        else:
            (qn_ref,) = o_refs
            zn_ref = None
        agg = agg_ref[...]
        zfull = z_ref[...]
        rows = (pl.program_id(0) * ROW_BLK
                + lax.broadcasted_iota(jnp.int32, (ROW_BLK, 1), 0))
        valid = jnp.broadcast_to(rows < N, (ROW_BLK, DIM))
        h = jnp.maximum(zfull + agg[0] + agg[1] + b1_ref[...], 0.0)
        h = jnp.where(valid, h, 0.0)
        xi = jnp.dot(h, w2_ref[...], preferred_element_type=jnp.float32)
        xi = jnp.where(valid, jnp.maximum(xi + b2_ref[...], 0.0), 0.0)
        qn_ref[...] = q_ref[...] + jnp.dot(
            xi, li_ref[...], preferred_element_type=jnp.float32)
        if has_next:
            zn_ref[...] = jnp.dot(xi, w1n_ref[...],
                                  preferred_element_type=jnp.float32)

    outs = pl.pallas_call(
        body,
        grid=(Z_ROWS // ROW_BLK,),
        in_specs=[
            pl.BlockSpec((ROW_BLK, DIM), lambda i: (i, 0)),
            pl.BlockSpec((NC, ROW_BLK, DIM), lambda i: (0, i, 0)),
            pl.BlockSpec((ROW_BLK, OUT), lambda i: (i, 0)),
            pl.BlockSpec((1, DIM), lambda i: (0, 0)),
            pl.BlockSpec((DIM, DIM), lambda i: (0, 0)),
            pl.BlockSpec((1, DIM), lambda i: (0, 0)),
            pl.BlockSpec((DIM, OUT), lambda i: (0, 0)),
            pl.BlockSpec((DIM, DIM), lambda i: (0, 0)),
        ],
        out_specs=out_specs,
        out_shape=out_shape,
    )(z, agg, q, b1.reshape(1, DIM), w2, b2.reshape(1, DIM), li, w1n)
    if has_next:
        return outs
    return None, outs[0]


_POOL_CHUNK = 1000


def _pool_body(q_ref, batch_ref, o_ref):
    def body(j, carry):
        s_acc, c_acc = carry
        brow = batch_ref[pl.ds(j, 1), :]
        gids = lax.broadcasted_iota(jnp.int32, (G, _POOL_CHUNK), 0)
        onehot_t = (jnp.broadcast_to(brow, (G, _POOL_CHUNK)) == gids
                    ).astype(jnp.float32)
        qc = q_ref[pl.ds(j * _POOL_CHUNK, _POOL_CHUNK), :]
        s_acc = s_acc + lax.dot_general(
            onehot_t, qc, (((1,), (0,)), ((), ())),
            preferred_element_type=jnp.float32)
        c_acc = c_acc + jnp.sum(onehot_t, axis=1, keepdims=True)
        return s_acc, c_acc

    s0 = jnp.zeros((G, OUT), jnp.float32)
    c0 = jnp.zeros((G, 1), jnp.float32)
    s_acc, c_acc = lax.fori_loop(0, N // _POOL_CHUNK, body, (s0, c0))
    o_ref[...] = jnp.tanh(s_acc / jnp.maximum(c_acc, 1.0))


def _tc_pool(q, batch2d):
    return pl.pallas_call(
        _pool_body,
        grid=(1,),
        in_specs=[
            pl.BlockSpec((N, OUT), lambda i: (0, 0)),
            pl.BlockSpec((N // _POOL_CHUNK, _POOL_CHUNK), lambda i: (0, 0)),
        ],
        out_specs=pl.BlockSpec((G, OUT), lambda i: (0, 0)),
        out_shape=jax.ShapeDtypeStruct((G, OUT), jnp.float32),
    )(q, batch2d)


def kernel(x, edge_index, batch,
           c1_W1, c1_b1, c1_W2, c1_b2,
           c2_W1, c2_b1, c2_W2, c2_b2,
           c3_W1, c3_b1, c3_W2, c3_b2,
           c4_W1, c4_b1, c4_W2, c4_b2,
           c5_W1, c5_b1, c5_W2, c5_b2,
           l1, l2, l3, l4, l5):
    src = edge_index[0].astype(jnp.int32)
    dst = edge_index[1].astype(jnp.int32)
    pad = E_PAD - E
    pad_ar = jnp.arange(pad, dtype=jnp.int32)
    src_p = jnp.concatenate([src, N + pad_ar % (Z_ROWS - N)])
    dst_p = jnp.concatenate([dst, (pad_ar * 131) % N])
    srcw = src_p.reshape(NW * NCHUNK, CH)
    dstw = dst_p.reshape(NW * NCHUNK, CH)
    zeros = jnp.zeros((ZSL, DIM), jnp.float32)
    batch2d = batch.astype(jnp.int32).reshape(N // _POOL_CHUNK, _POOL_CHUNK)
    xpad = jnp.pad(x, ((0, Z_ROWS - N), (0, 0)))

    q = jnp.zeros((Z_ROWS, OUT), jnp.float32)
    z = _tc_pre(xpad, c1_W1)
    params = [(c1_b1, c1_W2, c1_b2, l1, c2_W1),
              (c2_b1, c2_W2, c2_b2, l2, c3_W1),
              (c3_b1, c3_W2, c3_b2, l3, c4_W1),
              (c4_b1, c4_W2, c4_b2, l4, c5_W1),
              (c5_b1, c5_W2, c5_b2, l5, None)]
    for b1, w2, b2, li, w1n in params:
        agg = _sc_agg(z, srcw, dstw, zeros)
        z, q = _tc_layer(z, agg, q, b1, w2, b2, li, w1n)
    return _tc_pool(q[:N], batch2d)


# speedup vs baseline: 10.7526x; 1.0724x over previous
import functools

import jax
import jax.numpy as jnp
from jax import lax
from jax.experimental import pallas as pl
from jax.experimental.pallas import tpu as pltpu
from jax.experimental.pallas import tpu_sc as plsc

N = 10000
E = 320000
F_IN = 128
DIM = 64
OUT = 32
G = 128

NC = 2
NS = 16
NW = NC * NS
CH = 128
NCHUNK = 80
E_PAD = NW * NCHUNK * CH
Z_ROWS = 10016
ROW_BLK = 2504
ZSL = 632
ZSL_LAST = N - (NS - 1) * ZSL


def _sc_agg_body(z_hbm, srcw_hbm, dstw_hbm, zeros_hbm, out_hbm,
                 src_v, dst_v, row_a, row_b, row_c, row_d, acc,
                 sem_a, sem_b, sem_c, sem_d):
    c = lax.axis_index("c")
    s = lax.axis_index("s")
    w = c * NS + s

    pltpu.sync_copy(srcw_hbm.at[pl.ds(w * NCHUNK, NCHUNK)], src_v)
    pltpu.sync_copy(dstw_hbm.at[pl.ds(w * NCHUNK, NCHUNK)], dst_v)

    @pl.when(s < NS - 1)
    def _():
        pltpu.sync_copy(zeros_hbm, acc.at[pl.ds(s * ZSL, ZSL)])

    @pl.when(s == NS - 1)
    def _():
        pltpu.sync_copy(zeros_hbm.at[pl.ds(0, ZSL_LAST)],
                        acc.at[pl.ds((NS - 1) * ZSL, ZSL_LAST)])

    plsc.subcore_barrier()

    def body(i, carry):
        j = 4 * i
        cps = [pltpu.async_copy(z_hbm.at[src_v.at[j + k]], row, sem)
               for k, (row, sem) in enumerate(
                   ((row_a, sem_a), (row_b, sem_b),
                    (row_c, sem_c), (row_d, sem_d)))]
        for k, (cp, row) in enumerate(zip(cps, (row_a, row_b, row_c, row_d))):
            cp.wait()
            pltpu.sync_copy(row, acc.at[dst_v.at[j + k]], add=True)
        return carry

    lax.fori_loop(0, NCHUNK // 4, body, 0)
    plsc.subcore_barrier()

    @pl.when(s < NS - 1)
    def _():
        pltpu.sync_copy(acc.at[pl.ds(s * ZSL, ZSL)],
                        out_hbm.at[c, pl.ds(s * ZSL, ZSL)])

    @pl.when(s == NS - 1)
    def _():
        pltpu.sync_copy(acc.at[pl.ds((NS - 1) * ZSL, ZSL_LAST)],
                        out_hbm.at[c, pl.ds((NS - 1) * ZSL, ZSL_LAST)])


@functools.cache
def _sc_agg_kernel():
    return pl.kernel(
        _sc_agg_body,
        out_type=jax.ShapeDtypeStruct((NC, Z_ROWS, DIM), jnp.float32),
        mesh=plsc.VectorSubcoreMesh(core_axis_name="c", subcore_axis_name="s",
                                    num_cores=NC, num_subcores=NS),
        scratch_types=[
            pltpu.VMEM((NCHUNK, CH), jnp.int32),
            pltpu.VMEM((NCHUNK, CH), jnp.int32),
            pltpu.VMEM((CH, DIM), jnp.float32),
            pltpu.VMEM((CH, DIM), jnp.float32),
            pltpu.VMEM((CH, DIM), jnp.float32),
            pltpu.VMEM((CH, DIM), jnp.float32),
            pltpu.VMEM_SHARED((N, DIM), jnp.float32),
            pltpu.SemaphoreType.DMA,
            pltpu.SemaphoreType.DMA,
            pltpu.SemaphoreType.DMA,
            pltpu.SemaphoreType.DMA,
        ],
        compiler_params=pltpu.CompilerParams(use_tc_tiling_on_sc=False),
    )


def _sc_agg(z, srcw, dstw, zeros):
    return _sc_agg_kernel()(z, srcw, dstw, zeros)


def _pre_body(x_ref, w_ref, o_ref):
    o_ref[...] = jnp.dot(x_ref[...], w_ref[...],
                         preferred_element_type=jnp.float32)


def _tc_pre(x, w1):
    return pl.pallas_call(
        _pre_body,
        grid=(Z_ROWS // ROW_BLK,),
        in_specs=[
            pl.BlockSpec((ROW_BLK, F_IN), lambda i: (i, 0)),
            pl.BlockSpec((F_IN, DIM), lambda i: (0, 0)),
        ],
        out_specs=pl.BlockSpec((ROW_BLK, DIM), lambda i: (i, 0)),
        out_shape=jax.ShapeDtypeStruct((Z_ROWS, DIM), jnp.float32),
    )(x, w1)


def _tc_layer(z, agg, q, b1, w2, b2, li, w1n):
    has_next = w1n is not None
    if not has_next:
        w1n = w2
    out_shape = [jax.ShapeDtypeStruct((Z_ROWS, DIM), jnp.float32),
                 jax.ShapeDtypeStruct((Z_ROWS, OUT), jnp.float32)]
    out_specs = [pl.BlockSpec((ROW_BLK, DIM), lambda i: (i, 0)),
                 pl.BlockSpec((ROW_BLK, OUT), lambda i: (i, 0))]
    if not has_next:
        out_shape = out_shape[1:]
        out_specs = out_specs[1:]

    def body(z_ref, agg_ref, q_ref, b1_ref, w2_ref, b2_ref, li_ref, w1n_ref,
             *o_refs):
        if has_next:
            zn_ref, qn_ref = o_refs
        else:
            (qn_ref,) = o_refs
            zn_ref = None
        agg = agg_ref[...]
        zfull = z_ref[...]
        rows = (pl.program_id(0) * ROW_BLK
                + lax.broadcasted_iota(jnp.int32, (ROW_BLK, 1), 0))
        valid = jnp.broadcast_to(rows < N, (ROW_BLK, DIM))
        h = jnp.maximum(zfull + agg[0] + agg[1] + b1_ref[...], 0.0)
        h = jnp.where(valid, h, 0.0)
        xi = jnp.dot(h, w2_ref[...], preferred_element_type=jnp.float32)
        xi = jnp.where(valid, jnp.maximum(xi + b2_ref[...], 0.0), 0.0)
        qn_ref[...] = q_ref[...] + jnp.dot(
            xi, li_ref[...], preferred_element_type=jnp.float32)
        if has_next:
            zn_ref[...] = jnp.dot(xi, w1n_ref[...],
                                  preferred_element_type=jnp.float32)

    outs = pl.pallas_call(
        body,
        grid=(Z_ROWS // ROW_BLK,),
        in_specs=[
            pl.BlockSpec((ROW_BLK, DIM), lambda i: (i, 0)),
            pl.BlockSpec((NC, ROW_BLK, DIM), lambda i: (0, i, 0)),
            pl.BlockSpec((ROW_BLK, OUT), lambda i: (i, 0)),
            pl.BlockSpec((1, DIM), lambda i: (0, 0)),
            pl.BlockSpec((DIM, DIM), lambda i: (0, 0)),
            pl.BlockSpec((1, DIM), lambda i: (0, 0)),
            pl.BlockSpec((DIM, OUT), lambda i: (0, 0)),
            pl.BlockSpec((DIM, DIM), lambda i: (0, 0)),
        ],
        out_specs=out_specs,
        out_shape=out_shape,
    )(z, agg, q, b1.reshape(1, DIM), w2, b2.reshape(1, DIM), li, w1n)
    if has_next:
        return outs
    return None, outs[0]


_POOL_CHUNK = 1000


def _pool_body(q_ref, batch_ref, o_ref):
    def body(j, carry):
        s_acc, c_acc = carry
        brow = batch_ref[pl.ds(j, 1), :]
        gids = lax.broadcasted_iota(jnp.int32, (G, _POOL_CHUNK), 0)
        onehot_t = (jnp.broadcast_to(brow, (G, _POOL_CHUNK)) == gids
                    ).astype(jnp.float32)
        qc = q_ref[pl.ds(j * _POOL_CHUNK, _POOL_CHUNK), :]
        s_acc = s_acc + lax.dot_general(
            onehot_t, qc, (((1,), (0,)), ((), ())),
            preferred_element_type=jnp.float32)
        c_acc = c_acc + jnp.sum(onehot_t, axis=1, keepdims=True)
        return s_acc, c_acc

    s0 = jnp.zeros((G, OUT), jnp.float32)
    c0 = jnp.zeros((G, 1), jnp.float32)
    s_acc, c_acc = lax.fori_loop(0, N // _POOL_CHUNK, body, (s0, c0))
    o_ref[...] = jnp.tanh(s_acc / jnp.maximum(c_acc, 1.0))


def _tc_pool(q, batch2d):
    return pl.pallas_call(
        _pool_body,
        grid=(1,),
        in_specs=[
            pl.BlockSpec((N, OUT), lambda i: (0, 0)),
            pl.BlockSpec((N // _POOL_CHUNK, _POOL_CHUNK), lambda i: (0, 0)),
        ],
        out_specs=pl.BlockSpec((G, OUT), lambda i: (0, 0)),
        out_shape=jax.ShapeDtypeStruct((G, OUT), jnp.float32),
    )(q, batch2d)


def kernel(x, edge_index, batch,
           c1_W1, c1_b1, c1_W2, c1_b2,
           c2_W1, c2_b1, c2_W2, c2_b2,
           c3_W1, c3_b1, c3_W2, c3_b2,
           c4_W1, c4_b1, c4_W2, c4_b2,
           c5_W1, c5_b1, c5_W2, c5_b2,
           l1, l2, l3, l4, l5):
    src = edge_index[0].astype(jnp.int32)
    dst = edge_index[1].astype(jnp.int32)
    pad = E_PAD - E
    pad_ar = jnp.arange(pad, dtype=jnp.int32)
    src_p = jnp.concatenate([src, N + pad_ar % (Z_ROWS - N)])
    dst_p = jnp.concatenate([dst, (pad_ar * 131) % N])
    srcw = src_p.reshape(NW * NCHUNK, CH)
    dstw = dst_p.reshape(NW * NCHUNK, CH)
    zeros = jnp.zeros((ZSL, DIM), jnp.float32)
    batch2d = batch.astype(jnp.int32).reshape(N // _POOL_CHUNK, _POOL_CHUNK)
    xpad = jnp.pad(x, ((0, Z_ROWS - N), (0, 0)))

    q = jnp.zeros((Z_ROWS, OUT), jnp.float32)
    z = _tc_pre(xpad, c1_W1)
    params = [(c1_b1, c1_W2, c1_b2, l1, c2_W1),
              (c2_b1, c2_W2, c2_b2, l2, c3_W1),
              (c3_b1, c3_W2, c3_b2, l3, c4_W1),
              (c4_b1, c4_W2, c4_b2, l4, c5_W1),
              (c5_b1, c5_W2, c5_b2, l5, None)]
    for b1, w2, b2, li, w1n in params:
        agg = _sc_agg(z, srcw, dstw, zeros)
        z, q = _tc_layer(z, agg, q, b1, w2, b2, li, w1n)
    return _tc_pool(q[:N], batch2d)


# speedup vs baseline: 12.6380x; 1.1753x over previous
import functools

import jax
import jax.numpy as jnp
from jax import lax
from jax.experimental import pallas as pl
from jax.experimental.pallas import tpu as pltpu
from jax.experimental.pallas import tpu_sc as plsc

N = 10000
E = 320000
F_IN = 128
DIM = 64
OUT = 32
G = 128

NC = 2
NS = 16
NW = NC * NS
CH = 128
NCHUNK = 80
E_PAD = NW * NCHUNK * CH
Z_ROWS = 10016
ROW_BLK = 2504
ZSL = 632
ZSL_LAST = N - (NS - 1) * ZSL


def _sc_agg_body(z_hbm, srcw_hbm, dstw_hbm, zeros_hbm, out_hbm,
                 src_v, dst_v, row_a, row_b, row_c, row_d, acc,
                 sem_a, sem_b, sem_c, sem_d, ssem_a, ssem_b, ssem_c, ssem_d):
    c = lax.axis_index("c")
    s = lax.axis_index("s")
    w = c * NS + s

    pltpu.sync_copy(srcw_hbm.at[pl.ds(w * NCHUNK, NCHUNK)], src_v)
    pltpu.sync_copy(dstw_hbm.at[pl.ds(w * NCHUNK, NCHUNK)], dst_v)

    @pl.when(s < NS - 1)
    def _():
        pltpu.sync_copy(zeros_hbm, acc.at[pl.ds(s * ZSL, ZSL)])

    @pl.when(s == NS - 1)
    def _():
        pltpu.sync_copy(zeros_hbm.at[pl.ds(0, ZSL_LAST)],
                        acc.at[pl.ds((NS - 1) * ZSL, ZSL_LAST)])

    plsc.subcore_barrier()

    bufs = ((row_a, sem_a, ssem_a), (row_b, sem_b, ssem_b),
            (row_c, sem_c, ssem_c), (row_d, sem_d, ssem_d))

    for k, (row, gsem, _) in enumerate(bufs):
        pltpu.async_copy(z_hbm.at[src_v.at[k]], row, gsem)

    def body(i, carry):
        j = 4 * i
        for k, (row, gsem, ssem) in enumerate(bufs):
            pltpu.make_async_copy(z_hbm.at[src_v.at[j + k]], row, gsem).wait()
            pltpu.async_copy(row, acc.at[dst_v.at[j + k]], ssem, add=True)
        for k, (row, gsem, ssem) in enumerate(bufs):
            pltpu.make_async_copy(row, acc.at[dst_v.at[j + k]], ssem).wait()
            pltpu.async_copy(z_hbm.at[src_v.at[j + 4 + k]], row, gsem)
        return carry

    lax.fori_loop(0, NCHUNK // 4 - 1, body, 0)
    jlast = NCHUNK - 4
    for k, (row, gsem, ssem) in enumerate(bufs):
        pltpu.make_async_copy(z_hbm.at[src_v.at[jlast + k]], row, gsem).wait()
        pltpu.async_copy(row, acc.at[dst_v.at[jlast + k]], ssem, add=True)
    for k, (row, gsem, ssem) in enumerate(bufs):
        pltpu.make_async_copy(row, acc.at[dst_v.at[jlast + k]], ssem).wait()
    plsc.subcore_barrier()

    @pl.when(s < NS - 1)
    def _():
        pltpu.sync_copy(acc.at[pl.ds(s * ZSL, ZSL)],
                        out_hbm.at[c, pl.ds(s * ZSL, ZSL)])

    @pl.when(s == NS - 1)
    def _():
        pltpu.sync_copy(acc.at[pl.ds((NS - 1) * ZSL, ZSL_LAST)],
                        out_hbm.at[c, pl.ds((NS - 1) * ZSL, ZSL_LAST)])


@functools.cache
def _sc_agg_kernel():
    return pl.kernel(
        _sc_agg_body,
        out_type=jax.ShapeDtypeStruct((NC, Z_ROWS, DIM), jnp.float32),
        mesh=plsc.VectorSubcoreMesh(core_axis_name="c", subcore_axis_name="s",
                                    num_cores=NC, num_subcores=NS),
        scratch_types=[
            pltpu.VMEM((NCHUNK, CH), jnp.int32),
            pltpu.VMEM((NCHUNK, CH), jnp.int32),
            pltpu.VMEM((CH, DIM), jnp.float32),
            pltpu.VMEM((CH, DIM), jnp.float32),
            pltpu.VMEM((CH, DIM), jnp.float32),
            pltpu.VMEM((CH, DIM), jnp.float32),
            pltpu.VMEM_SHARED((N, DIM), jnp.float32),
            pltpu.SemaphoreType.DMA,
            pltpu.SemaphoreType.DMA,
            pltpu.SemaphoreType.DMA,
            pltpu.SemaphoreType.DMA,
            pltpu.SemaphoreType.DMA,
            pltpu.SemaphoreType.DMA,
            pltpu.SemaphoreType.DMA,
            pltpu.SemaphoreType.DMA,
        ],
        compiler_params=pltpu.CompilerParams(use_tc_tiling_on_sc=False),
    )


def _sc_agg(z, srcw, dstw, zeros):
    return _sc_agg_kernel()(z, srcw, dstw, zeros)


def _pre_body(x_ref, w_ref, o_ref):
    o_ref[...] = jnp.dot(x_ref[...], w_ref[...],
                         preferred_element_type=jnp.float32)


def _tc_pre(x, w1):
    return pl.pallas_call(
        _pre_body,
        grid=(Z_ROWS // ROW_BLK,),
        in_specs=[
            pl.BlockSpec((ROW_BLK, F_IN), lambda i: (i, 0)),
            pl.BlockSpec((F_IN, DIM), lambda i: (0, 0)),
        ],
        out_specs=pl.BlockSpec((ROW_BLK, DIM), lambda i: (i, 0)),
        out_shape=jax.ShapeDtypeStruct((Z_ROWS, DIM), jnp.float32),
    )(x, w1)


def _tc_layer(z, agg, q, b1, w2, b2, li, w1n):
    has_next = w1n is not None
    if not has_next:
        w1n = w2
    out_shape = [jax.ShapeDtypeStruct((Z_ROWS, DIM), jnp.float32),
                 jax.ShapeDtypeStruct((Z_ROWS, OUT), jnp.float32)]
    out_specs = [pl.BlockSpec((ROW_BLK, DIM), lambda i: (i, 0)),
                 pl.BlockSpec((ROW_BLK, OUT), lambda i: (i, 0))]
    if not has_next:
        out_shape = out_shape[1:]
        out_specs = out_specs[1:]

    def body(z_ref, agg_ref, q_ref, b1_ref, w2_ref, b2_ref, li_ref, w1n_ref,
             *o_refs):
        if has_next:
            zn_ref, qn_ref = o_refs
        else:
            (qn_ref,) = o_refs
            zn_ref = None
        agg = agg_ref[...]
        zfull = z_ref[...]
        rows = (pl.program_id(0) * ROW_BLK
                + lax.broadcasted_iota(jnp.int32, (ROW_BLK, 1), 0))
        valid = jnp.broadcast_to(rows < N, (ROW_BLK, DIM))
        h = jnp.maximum(zfull + agg[0] + agg[1] + b1_ref[...], 0.0)
        h = jnp.where(valid, h, 0.0)
        xi = jnp.dot(h, w2_ref[...], preferred_element_type=jnp.float32)
        xi = jnp.where(valid, jnp.maximum(xi + b2_ref[...], 0.0), 0.0)
        qn_ref[...] = q_ref[...] + jnp.dot(
            xi, li_ref[...], preferred_element_type=jnp.float32)
        if has_next:
            zn_ref[...] = jnp.dot(xi, w1n_ref[...],
                                  preferred_element_type=jnp.float32)

    outs = pl.pallas_call(
        body,
        grid=(Z_ROWS // ROW_BLK,),
        in_specs=[
            pl.BlockSpec((ROW_BLK, DIM), lambda i: (i, 0)),
            pl.BlockSpec((NC, ROW_BLK, DIM), lambda i: (0, i, 0)),
            pl.BlockSpec((ROW_BLK, OUT), lambda i: (i, 0)),
            pl.BlockSpec((1, DIM), lambda i: (0, 0)),
            pl.BlockSpec((DIM, DIM), lambda i: (0, 0)),
            pl.BlockSpec((1, DIM), lambda i: (0, 0)),
            pl.BlockSpec((DIM, OUT), lambda i: (0, 0)),
            pl.BlockSpec((DIM, DIM), lambda i: (0, 0)),
        ],
        out_specs=out_specs,
        out_shape=out_shape,
    )(z, agg, q, b1.reshape(1, DIM), w2, b2.reshape(1, DIM), li, w1n)
    if has_next:
        return outs
    return None, outs[0]


_POOL_CHUNK = 1000


def _pool_body(q_ref, batch_ref, o_ref):
    def body(j, carry):
        s_acc, c_acc = carry
        brow = batch_ref[pl.ds(j, 1), :]
        gids = lax.broadcasted_iota(jnp.int32, (G, _POOL_CHUNK), 0)
        onehot_t = (jnp.broadcast_to(brow, (G, _POOL_CHUNK)) == gids
                    ).astype(jnp.float32)
        qc = q_ref[pl.ds(j * _POOL_CHUNK, _POOL_CHUNK), :]
        s_acc = s_acc + lax.dot_general(
            onehot_t, qc, (((1,), (0,)), ((), ())),
            preferred_element_type=jnp.float32)
        c_acc = c_acc + jnp.sum(onehot_t, axis=1, keepdims=True)
        return s_acc, c_acc

    s0 = jnp.zeros((G, OUT), jnp.float32)
    c0 = jnp.zeros((G, 1), jnp.float32)
    s_acc, c_acc = lax.fori_loop(0, N // _POOL_CHUNK, body, (s0, c0))
    o_ref[...] = jnp.tanh(s_acc / jnp.maximum(c_acc, 1.0))


def _tc_pool(q, batch2d):
    return pl.pallas_call(
        _pool_body,
        grid=(1,),
        in_specs=[
            pl.BlockSpec((N, OUT), lambda i: (0, 0)),
            pl.BlockSpec((N // _POOL_CHUNK, _POOL_CHUNK), lambda i: (0, 0)),
        ],
        out_specs=pl.BlockSpec((G, OUT), lambda i: (0, 0)),
        out_shape=jax.ShapeDtypeStruct((G, OUT), jnp.float32),
    )(q, batch2d)


def kernel(x, edge_index, batch,
           c1_W1, c1_b1, c1_W2, c1_b2,
           c2_W1, c2_b1, c2_W2, c2_b2,
           c3_W1, c3_b1, c3_W2, c3_b2,
           c4_W1, c4_b1, c4_W2, c4_b2,
           c5_W1, c5_b1, c5_W2, c5_b2,
           l1, l2, l3, l4, l5):
    src = edge_index[0].astype(jnp.int32)
    dst = edge_index[1].astype(jnp.int32)
    pad = E_PAD - E
    pad_ar = jnp.arange(pad, dtype=jnp.int32)
    src_p = jnp.concatenate([src, N + pad_ar % (Z_ROWS - N)])
    dst_p = jnp.concatenate([dst, (pad_ar * 131) % N])
    srcw = src_p.reshape(NW * NCHUNK, CH)
    dstw = dst_p.reshape(NW * NCHUNK, CH)
    zeros = jnp.zeros((ZSL, DIM), jnp.float32)
    batch2d = batch.astype(jnp.int32).reshape(N // _POOL_CHUNK, _POOL_CHUNK)
    xpad = jnp.pad(x, ((0, Z_ROWS - N), (0, 0)))

    q = jnp.zeros((Z_ROWS, OUT), jnp.float32)
    z = _tc_pre(xpad, c1_W1)
    params = [(c1_b1, c1_W2, c1_b2, l1, c2_W1),
              (c2_b1, c2_W2, c2_b2, l2, c3_W1),
              (c3_b1, c3_W2, c3_b2, l3, c4_W1),
              (c4_b1, c4_W2, c4_b2, l4, c5_W1),
              (c5_b1, c5_W2, c5_b2, l5, None)]
    for b1, w2, b2, li, w1n in params:
        agg = _sc_agg(z, srcw, dstw, zeros)
        z, q = _tc_layer(z, agg, q, b1, w2, b2, li, w1n)
    return _tc_pool(q[:N], batch2d)
